# baseline (speedup 1.0000x reference)
import functools

import jax
import jax.numpy as jnp
from jax import lax
from jax.experimental import pallas as pl
from jax.experimental.pallas import tpu as pltpu
from jax.experimental.pallas import tpu_sc as plsc

N = 16384
D = 512
K = 8192
NBANK = 1000
NBP = 1024
GAMMA = 0.7
TAU = 0.1
EPS = 1e-8

BM = 256
BK = 256


def _rownorm(x):
    return x / jnp.maximum(
        jnp.sqrt(jnp.sum(x * x, axis=1, keepdims=True)), EPS)


def _argmin_body(tf_ref, cb_ref, idx_ref, idxc_ref):
    tf = tf_ref[...]
    cb = cb_ref[...]
    dot = lax.dot_general(tf, cb, (((1,), (1,)), ((), ())),
                          preferred_element_type=jnp.float32)
    tf2 = jnp.sum(tf * tf, axis=1, keepdims=True)
    cb2 = jnp.sum(cb * cb, axis=1)[None, :]
    d2 = (tf2 - 2.0 * dot) + cb2
    idx = jnp.argmin(d2, axis=1).astype(jnp.int32)
    idx_ref[...] = idx
    idxc_ref[...] = jnp.minimum(idx, NBANK - 1)


def _vq_argmin(tf, cb):
    return pl.pallas_call(
        _argmin_body,
        grid=(N // BM,),
        in_specs=[
            pl.BlockSpec((BM, D), lambda i: (i, 0)),
            pl.BlockSpec((K, D), lambda i: (0, 0)),
        ],
        out_specs=[
            pl.BlockSpec((BM,), lambda i: (i,)),
            pl.BlockSpec((BM,), lambda i: (i,)),
        ],
        out_shape=[
            jax.ShapeDtypeStruct((N,), jnp.int32),
            jax.ShapeDtypeStruct((N,), jnp.int32),
        ],
    )(tf, cb)


def _vtable_body(cb_ref, vw_ref, out_ref):
    cbn = _rownorm(cb_ref[...])
    vwn = _rownorm(vw_ref[...])
    out_ref[...] = GAMMA * lax.dot_general(
        cbn, vwn, (((1,), (1,)), ((), ())),
        preferred_element_type=jnp.float32)


def _vtable(cb, vw_pad):
    return pl.pallas_call(
        _vtable_body,
        grid=(K // BK,),
        in_specs=[
            pl.BlockSpec((BK, D), lambda i: (i, 0)),
            pl.BlockSpec((NBP, D), lambda i: (0, 0)),
        ],
        out_specs=pl.BlockSpec((BK, NBP), lambda i: (i, 0)),
        out_shape=jax.ShapeDtypeStruct((K, NBP), jnp.float32),
    )(cb, vw_pad)


def _sc_info():
    try:
        info = plsc.get_sparse_core_info()
        return info.num_cores, info.num_subcores
    except Exception:
        return 2, 16


def _sc_gather(table, idx, b_chunk):
    v_rows, d_row = table.shape
    (b,) = idx.shape
    nc, ns = _sc_info()
    nw = nc * ns
    b_per_w = b // nw
    assert b % (8 * nw) == 0 and b_per_w % b_chunk == 0
    n_chunks = b_per_w // b_chunk
    mesh = plsc.VectorSubcoreMesh(core_axis_name="c", subcore_axis_name="s",
                                  num_cores=nc, num_subcores=ns)

    @functools.partial(
        pl.kernel, mesh=mesh,
        out_type=jax.ShapeDtypeStruct((b, d_row), jnp.float32),
        scratch_types=[
            pltpu.VMEM((2, b_chunk), jnp.int32),
            pltpu.VMEM((2, b_chunk, d_row), jnp.float32),
            pltpu.SemaphoreType.DMA,
            pltpu.SemaphoreType.DMA,
        ],
    )
    def k(table_hbm, idx_hbm, out_hbm, idx_v, rows_v, sem0, sem1):
        wid = lax.axis_index("s") * nc + lax.axis_index("c")
        base = wid * b_per_w
        sems = (sem0, sem1)
        pltpu.sync_copy(idx_hbm.at[pl.ds(base, b_chunk)], idx_v.at[0])
        cp0 = pltpu.async_copy(table_hbm.at[idx_v.at[0]], rows_v.at[0], sem0)
        copies = [cp0, None]
        for j in range(n_chunks):
            s = j % 2
            nxt = (j + 1) % 2
            if j + 1 < n_chunks:
                pltpu.sync_copy(
                    idx_hbm.at[pl.ds(base + (j + 1) * b_chunk, b_chunk)],
                    idx_v.at[nxt])
                copies[nxt] = pltpu.async_copy(
                    table_hbm.at[idx_v.at[nxt]], rows_v.at[nxt], sems[nxt])
            copies[s].wait()
            pltpu.sync_copy(
                rows_v.at[s],
                out_hbm.at[pl.ds(base + j * b_chunk, b_chunk)])

    return k(table, idx)


def _attn_body(pce_ref, vrows_ref, tf_ref, prom_ref, ce_ref,
               out1_ref, out2_ref):
    cen = _rownorm(ce_ref[...])
    pcen = _rownorm(pce_ref[...])
    sem = lax.dot_general(pcen, cen, (((1,), (1,)), ((), ())),
                          preferred_element_type=jnp.float32)
    comb = vrows_ref[...] + (1.0 - GAMMA) * sem
    col = lax.broadcasted_iota(jnp.int32, (BM, NBP), 1)
    logits = jnp.where(col < NBANK, comb / TAU, -1e30)
    m = jnp.max(logits, axis=1, keepdims=True)
    p = jnp.exp(logits - m)
    p = p / jnp.sum(p, axis=1, keepdims=True)
    out1_ref[...] = lax.dot_general(p, ce_ref[...], (((1,), (0,)), ((), ())),
                                    preferred_element_type=jnp.float32)
    out2_ref[...] = tf_ref[...] + prom_ref[...]


def _attn_tail(pce, vrows, tf, prom, ce_pad):
    return pl.pallas_call(
        _attn_body,
        grid=(N // BM,),
        in_specs=[
            pl.BlockSpec((BM, D), lambda i: (i, 0)),
            pl.BlockSpec((BM, NBP), lambda i: (i, 0)),
            pl.BlockSpec((BM, D), lambda i: (i, 0)),
            pl.BlockSpec((BM, D), lambda i: (i, 0)),
            pl.BlockSpec((NBP, D), lambda i: (0, 0)),
        ],
        out_specs=[
            pl.BlockSpec((BM, D), lambda i: (i, 0)),
            pl.BlockSpec((BM, D), lambda i: (i, 0)),
        ],
        out_shape=[
            jax.ShapeDtypeStruct((N, D), jnp.float32),
            jax.ShapeDtypeStruct((N, D), jnp.float32),
        ],
    )(pce, vrows, tf, prom, ce_pad)


def kernel(test_features, predicted_class_emb, vqgan_codebook,
           token_prompts, visual_words, class_embeddings):
    vw_pad = jnp.pad(visual_words, ((0, NBP - NBANK), (0, 0)))
    ce_pad = jnp.pad(class_embeddings, ((0, NBP - NBANK), (0, 0)))

    tfs = jax.lax.stop_gradient(test_features)
    cbs = jax.lax.stop_gradient(vqgan_codebook)
    d2 = (jnp.sum(tfs * tfs, axis=1)[:, None]
          - 2.0 * (tfs @ cbs.T)
          + jnp.sum(cbs * cbs, axis=1)[None, :])
    idx = jnp.argmin(d2, axis=1).astype(jnp.int32)
    idx_clamped = jnp.minimum(idx, NBANK - 1)
    vtab = _vtable(vqgan_codebook, vw_pad)

    prompts = _sc_gather(token_prompts, idx_clamped, b_chunk=64)
    vrows = _sc_gather(vtab, idx, b_chunk=32)

    cache_logits, augmented = _attn_tail(
        predicted_class_emb, vrows, test_features, prompts, ce_pad)
    return (cache_logits, augmented)


# speedup vs baseline: 1.0260x; 1.0260x over previous
import functools

import jax
import jax.numpy as jnp
from jax import lax
from jax.experimental import pallas as pl
from jax.experimental.pallas import tpu as pltpu
from jax.experimental.pallas import tpu_sc as plsc

N = 16384
D = 512
K = 8192
NBANK = 1000
NBP = 1024
GAMMA = 0.7
TAU = 0.1
EPS = 1e-8

BM = 256
BK = 256


def _rownorm(x):
    return x / jnp.maximum(
        jnp.sqrt(jnp.sum(x * x, axis=1, keepdims=True)), EPS)


def _argmin_body(tf_ref, cb_ref, idx_ref, idxc_ref):
    tf = tf_ref[...]
    cb = cb_ref[...]
    dot = lax.dot_general(tf, cb, (((1,), (1,)), ((), ())),
                          preferred_element_type=jnp.float32)
    tf2 = jnp.sum(tf * tf, axis=1, keepdims=True)
    cb2 = jnp.sum(cb * cb, axis=1)[None, :]
    d2 = (tf2 - 2.0 * dot) + cb2
    idx = jnp.argmin(d2, axis=1).astype(jnp.int32)
    idx_ref[...] = idx
    idxc_ref[...] = jnp.minimum(idx, NBANK - 1)


def _vq_argmin(tf, cb):
    return pl.pallas_call(
        _argmin_body,
        grid=(N // BM,),
        in_specs=[
            pl.BlockSpec((BM, D), lambda i: (i, 0)),
            pl.BlockSpec((K, D), lambda i: (0, 0)),
        ],
        out_specs=[
            pl.BlockSpec((BM,), lambda i: (i,)),
            pl.BlockSpec((BM,), lambda i: (i,)),
        ],
        out_shape=[
            jax.ShapeDtypeStruct((N,), jnp.int32),
            jax.ShapeDtypeStruct((N,), jnp.int32),
        ],
    )(tf, cb)


def _vtable_body(cb_ref, vw_ref, out_ref):
    cbn = _rownorm(cb_ref[...])
    vwn = _rownorm(vw_ref[...])
    out_ref[...] = GAMMA * lax.dot_general(
        cbn, vwn, (((1,), (1,)), ((), ())),
        preferred_element_type=jnp.float32)


def _vtable(cb, vw_pad):
    return pl.pallas_call(
        _vtable_body,
        grid=(K // BK,),
        in_specs=[
            pl.BlockSpec((BK, D), lambda i: (i, 0)),
            pl.BlockSpec((NBP, D), lambda i: (0, 0)),
        ],
        out_specs=pl.BlockSpec((BK, NBP), lambda i: (i, 0)),
        out_shape=jax.ShapeDtypeStruct((K, NBP), jnp.float32),
    )(cb, vw_pad)


def _sc_info():
    try:
        info = plsc.get_sparse_core_info()
        return info.num_cores, info.num_subcores
    except Exception:
        return 2, 16


def _sc_gather(table, idx, b_chunk):
    v_rows, d_row = table.shape
    (b,) = idx.shape
    nc, ns = _sc_info()
    nw = nc * ns
    b_per_w = b // nw
    assert b % (8 * nw) == 0 and b_per_w % b_chunk == 0
    n_chunks = b_per_w // b_chunk
    mesh = plsc.VectorSubcoreMesh(core_axis_name="c", subcore_axis_name="s",
                                  num_cores=nc, num_subcores=ns)

    @functools.partial(
        pl.kernel, mesh=mesh,
        out_type=jax.ShapeDtypeStruct((b, d_row), jnp.float32),
        scratch_types=[
            pltpu.VMEM((2, b_chunk), jnp.int32),
            pltpu.VMEM((2, b_chunk, d_row), jnp.float32),
            pltpu.SemaphoreType.DMA,
            pltpu.SemaphoreType.DMA,
        ],
    )
    def k(table_hbm, idx_hbm, out_hbm, idx_v, rows_v, sem0, sem1):
        wid = lax.axis_index("s") * nc + lax.axis_index("c")
        base = wid * b_per_w
        sems = (sem0, sem1)
        pltpu.sync_copy(idx_hbm.at[pl.ds(base, b_chunk)], idx_v.at[0])
        cp0 = pltpu.async_copy(table_hbm.at[idx_v.at[0]], rows_v.at[0], sem0)
        copies = [cp0, None]
        for j in range(n_chunks):
            s = j % 2
            nxt = (j + 1) % 2
            if j + 1 < n_chunks:
                pltpu.sync_copy(
                    idx_hbm.at[pl.ds(base + (j + 1) * b_chunk, b_chunk)],
                    idx_v.at[nxt])
                copies[nxt] = pltpu.async_copy(
                    table_hbm.at[idx_v.at[nxt]], rows_v.at[nxt], sems[nxt])
            copies[s].wait()
            pltpu.sync_copy(
                rows_v.at[s],
                out_hbm.at[pl.ds(base + j * b_chunk, b_chunk)])

    return k(table, idx)


def _attn_body(pce_ref, tw_ref, vw_ref, tf_ref, prom_ref, ce_ref,
               out1_ref, out2_ref):
    cen = _rownorm(ce_ref[...])
    pcen = _rownorm(pce_ref[...])
    sem = lax.dot_general(pcen, cen, (((1,), (1,)), ((), ())),
                          preferred_element_type=jnp.float32)
    vwn = _rownorm(vw_ref[...])
    twn = _rownorm(tw_ref[...])
    vis = lax.dot_general(twn, vwn, (((1,), (1,)), ((), ())),
                          preferred_element_type=jnp.float32)
    comb = GAMMA * vis + (1.0 - GAMMA) * sem
    col = lax.broadcasted_iota(jnp.int32, (BM, NBP), 1)
    logits = jnp.where(col < NBANK, comb / TAU, -1e30)
    m = jnp.max(logits, axis=1, keepdims=True)
    p = jnp.exp(logits - m)
    p = p / jnp.sum(p, axis=1, keepdims=True)
    out1_ref[...] = lax.dot_general(p, ce_ref[...], (((1,), (0,)), ((), ())),
                                    preferred_element_type=jnp.float32)
    out2_ref[...] = tf_ref[...] + prom_ref[...]


def _attn_tail(pce, tw, vw_pad, tf, prom, ce_pad):
    return pl.pallas_call(
        _attn_body,
        grid=(N // BM,),
        in_specs=[
            pl.BlockSpec((BM, D), lambda i: (i, 0)),
            pl.BlockSpec((BM, D), lambda i: (i, 0)),
            pl.BlockSpec((NBP, D), lambda i: (0, 0)),
            pl.BlockSpec((BM, D), lambda i: (i, 0)),
            pl.BlockSpec((BM, D), lambda i: (i, 0)),
            pl.BlockSpec((NBP, D), lambda i: (0, 0)),
        ],
        out_specs=[
            pl.BlockSpec((BM, D), lambda i: (i, 0)),
            pl.BlockSpec((BM, D), lambda i: (i, 0)),
        ],
        out_shape=[
            jax.ShapeDtypeStruct((N, D), jnp.float32),
            jax.ShapeDtypeStruct((N, D), jnp.float32),
        ],
    )(pce, tw, vw_pad, tf, prom, ce_pad)


def kernel(test_features, predicted_class_emb, vqgan_codebook,
           token_prompts, visual_words, class_embeddings):
    vw_pad = jnp.pad(visual_words, ((0, NBP - NBANK), (0, 0)))
    ce_pad = jnp.pad(class_embeddings, ((0, NBP - NBANK), (0, 0)))

    tfs = jax.lax.stop_gradient(test_features)
    cbs = jax.lax.stop_gradient(vqgan_codebook)
    d2 = (jnp.sum(tfs * tfs, axis=1)[:, None]
          - 2.0 * (tfs @ cbs.T)
          + jnp.sum(cbs * cbs, axis=1)[None, :])
    idx = jnp.argmin(d2, axis=1).astype(jnp.int32)
    idx_clamped = jnp.minimum(idx, NBANK - 1)

    prompts = _sc_gather(token_prompts, idx_clamped, b_chunk=64)
    test_words = _sc_gather(vqgan_codebook, idx, b_chunk=64)

    cache_logits, augmented = _attn_tail(
        predicted_class_emb, test_words, vw_pad, test_features, prompts,
        ce_pad)
    return (cache_logits, augmented)


# speedup vs baseline: 1.0271x; 1.0011x over previous
import functools

import jax
import jax.numpy as jnp
from jax import lax
from jax.experimental import pallas as pl
from jax.experimental.pallas import tpu as pltpu
from jax.experimental.pallas import tpu_sc as plsc

N = 16384
D = 512
K = 8192
NBANK = 1000
NBP = 1024
GAMMA = 0.7
TAU = 0.1
EPS = 1e-8

BM = 256
BK = 256


def _rownorm(x):
    return x / jnp.maximum(
        jnp.sqrt(jnp.sum(x * x, axis=1, keepdims=True)), EPS)


def _sc_info():
    try:
        info = plsc.get_sparse_core_info()
        return info.num_cores, info.num_subcores
    except Exception:
        return 2, 16


def _sc_gather(table, idx, b_chunk):
    v_rows, d_row = table.shape
    (b,) = idx.shape
    nc, ns = _sc_info()
    nw = nc * ns
    b_per_w = b // nw
    assert b % (8 * nw) == 0 and b_per_w % b_chunk == 0
    n_chunks = b_per_w // b_chunk
    mesh = plsc.VectorSubcoreMesh(core_axis_name="c", subcore_axis_name="s",
                                  num_cores=nc, num_subcores=ns)

    @functools.partial(
        pl.kernel, mesh=mesh,
        out_type=jax.ShapeDtypeStruct((b, d_row), jnp.float32),
        scratch_types=[
            pltpu.VMEM((2, b_chunk), jnp.int32),
            pltpu.VMEM((2, b_chunk, d_row), jnp.float32),
            pltpu.SemaphoreType.DMA,
            pltpu.SemaphoreType.DMA,
        ],
    )
    def k(table_hbm, idx_hbm, out_hbm, idx_v, rows_v, sem0, sem1):
        wid = lax.axis_index("s") * nc + lax.axis_index("c")
        base = wid * b_per_w
        sems = (sem0, sem1)
        pltpu.sync_copy(idx_hbm.at[pl.ds(base, b_chunk)], idx_v.at[0])
        cp0 = pltpu.async_copy(table_hbm.at[idx_v.at[0]], rows_v.at[0], sem0)
        copies = [cp0, None]
        for j in range(n_chunks):
            s = j % 2
            nxt = (j + 1) % 2
            if j + 1 < n_chunks:
                pltpu.sync_copy(
                    idx_hbm.at[pl.ds(base + (j + 1) * b_chunk, b_chunk)],
                    idx_v.at[nxt])
                copies[nxt] = pltpu.async_copy(
                    table_hbm.at[idx_v.at[nxt]], rows_v.at[nxt], sems[nxt])
            copies[s].wait()
            pltpu.sync_copy(
                rows_v.at[s],
                out_hbm.at[pl.ds(base + j * b_chunk, b_chunk)])

    return k(table, idx)


def _attn_body(pce_ref, tw_ref, vw_ref, tf_ref, prom_ref, ce_ref,
               out1_ref, out2_ref):
    cen = _rownorm(ce_ref[...])
    pcen = _rownorm(pce_ref[...])
    sem = lax.dot_general(pcen, cen, (((1,), (1,)), ((), ())),
                          preferred_element_type=jnp.float32)
    vwn = _rownorm(vw_ref[...])
    twn = _rownorm(tw_ref[...])
    vis = lax.dot_general(twn, vwn, (((1,), (1,)), ((), ())),
                          preferred_element_type=jnp.float32)
    comb = GAMMA * vis + (1.0 - GAMMA) * sem
    col = lax.broadcasted_iota(jnp.int32, (BM, NBP), 1)
    logits = jnp.where(col < NBANK, comb / TAU, -1e30)
    m = jnp.max(logits, axis=1, keepdims=True)
    p = jnp.exp(logits - m)
    p = p / jnp.sum(p, axis=1, keepdims=True)
    out1_ref[...] = lax.dot_general(p, ce_ref[...], (((1,), (0,)), ((), ())),
                                    preferred_element_type=jnp.float32)
    out2_ref[...] = tf_ref[...] + prom_ref[...]


def _attn_tail(pce, tw, vw_pad, tf, prom, ce_pad):
    return pl.pallas_call(
        _attn_body,
        grid=(N // BM,),
        in_specs=[
            pl.BlockSpec((BM, D), lambda i: (i, 0)),
            pl.BlockSpec((BM, D), lambda i: (i, 0)),
            pl.BlockSpec((NBP, D), lambda i: (0, 0)),
            pl.BlockSpec((BM, D), lambda i: (i, 0)),
            pl.BlockSpec((BM, D), lambda i: (i, 0)),
            pl.BlockSpec((NBP, D), lambda i: (0, 0)),
        ],
        out_specs=[
            pl.BlockSpec((BM, D), lambda i: (i, 0)),
            pl.BlockSpec((BM, D), lambda i: (i, 0)),
        ],
        out_shape=[
            jax.ShapeDtypeStruct((N, D), jnp.float32),
            jax.ShapeDtypeStruct((N, D), jnp.float32),
        ],
    )(pce, tw, vw_pad, tf, prom, ce_pad)


def kernel(test_features, predicted_class_emb, vqgan_codebook,
           token_prompts, visual_words, class_embeddings):
    vw_pad = jnp.pad(visual_words, ((0, NBP - NBANK), (0, 0)))
    ce_pad = jnp.pad(class_embeddings, ((0, NBP - NBANK), (0, 0)))

    tfs = jax.lax.stop_gradient(test_features)
    cbs = jax.lax.stop_gradient(vqgan_codebook)
    d2 = (jnp.sum(tfs * tfs, axis=1)[:, None]
          - 2.0 * (tfs @ cbs.T)
          + jnp.sum(cbs * cbs, axis=1)[None, :])
    idx = jnp.argmin(d2, axis=1).astype(jnp.int32)
    idx_clamped = jnp.minimum(idx, NBANK - 1)

    prompts = _sc_gather(token_prompts, idx_clamped, b_chunk=64)
    test_words = _sc_gather(vqgan_codebook, idx, b_chunk=64)

    cache_logits, augmented = _attn_tail(
        predicted_class_emb, test_words, vw_pad, test_features, prompts,
        ce_pad)
    return (cache_logits, augmented)
